# Initial kernel scaffold; baseline (speedup 1.0000x reference)
import jax
import jax.numpy as jnp
from jax.experimental import pallas as pl


def kernel(events, lengths):
    raise NotImplementedError("write your pallas kernel here")


# speedup vs baseline: 283.5930x; 283.5930x over previous
import functools

import jax
import jax.numpy as jnp
import numpy as np
from jax import lax
from jax.experimental import pallas as pl
from jax.experimental.pallas import tpu as pltpu
from jax.experimental.pallas import tpu_sc as plsc

H, W = 240, 304
K = 10
R = 3
TAU = 1000000.0
DELTA_T = 100000.0
GH = (H + K - 1) // K
GW = (W + K - 1) // K
NC = GH * GW
S = 2 * R + 1
B = 8
TPAD = 2048

NGROUPS = 4
CPG = NC // NGROUPS
BIN = 64
CNT_LANE = 49
GWORDS = CPG * 2 * K * K
HWORDS = CPG * 2 * BIN
WCAP = TPAD + 16


def _body(ev_hbm, len_hbm, out_hbm, ev_v, len_v, g_v, h_v,
          wt, we, wg, wh, wly, wlx):
    ci = lax.axis_index("c")
    si = lax.axis_index("s")
    wid = si * 2 + ci
    b = wid // NGROUPS
    grp = wid % NGROUPS
    lo = grp * CPG

    pltpu.sync_copy(ev_hbm.at[b], ev_v)
    pltpu.sync_copy(len_hbm, len_v)

    iota16 = lax.iota(jnp.int32, 16)
    lenvec = len_v[pl.ds(0, 16)]
    length = jnp.sum(jnp.where(iota16 == b, lenvec, 0))

    def sload(ref, idx):
        return ref[pl.ds(idx, 16)][0]

    zeros16 = (iota16 * 0).astype(jnp.float32)
    ones16 = zeros16 + 1.0

    def zero_g(i, c):
        g_v[pl.ds(i * 16, 16)] = zeros16
        return c

    lax.fori_loop(0, GWORDS // 16, zero_g, 0)

    def zero_h(i, c):
        h_v[pl.ds(i * 16, 16)] = zeros16
        return c

    lax.fori_loop(0, HWORDS // 16, zero_h, 0)

    lane0 = iota16 == 0

    dy_t, dx_t, off_t = [], [], []
    for k in range(4):
        s = iota16 + 16 * k
        in49 = s < 49
        dy = jnp.where(in49, lax.div(s, S), 99)
        dx = lax.rem(s, S)
        dy_t.append(dy)
        dx_t.append(dx)
        off_t.append(jnp.where(in49, (lax.div(s, S) - R) * K + (dx - R), 0))
    cnt_t = ((iota16 + 48) == CNT_LANE).astype(jnp.float32)

    def scan(k, off):
        xi = ev_v[0, pl.ds(k * 16, 16)].astype(jnp.int32)
        yi = ev_v[1, pl.ds(k * 16, 16)].astype(jnp.int32)
        tv = ev_v[2, pl.ds(k * 16, 16)]
        pi = ev_v[3, pl.ds(k * 16, 16)].astype(jnp.int32)
        ch = lax.div(yi, K)
        cw = lax.div(xi, K)
        cid = ch * GW + cw
        lyv = yi - ch * K
        lxv = xi - cw * K
        idxv = k * 16 + iota16
        m = (idxv < length) & (cid >= lo) & (cid < lo + CPG)
        ev_exp = jnp.exp(tv * (1.0 / TAU))
        lcell = cid - lo
        gidx = (lcell * 2 + pi) * (K * K) + lyv * K + lxv
        hbase = (lcell * 2 + pi) * BIN
        sl = pl.ds(off, 16)
        plsc.store_compressed(wt.at[sl], tv, mask=m)
        plsc.store_compressed(we.at[sl], ev_exp, mask=m)
        plsc.store_compressed(wg.at[sl], gidx, mask=m)
        plsc.store_compressed(wh.at[sl], hbase, mask=m)
        plsc.store_compressed(wly.at[sl], lyv, mask=m)
        plsc.store_compressed(wlx.at[sl], lxv, mask=m)
        return off + plsc.all_reduce_population_count(m)[0]

    nw = lax.fori_loop(0, TPAD // 16, scan, jnp.int32(0))

    def proc(m_i, L):
        ti = sload(wt, m_i)
        ei = sload(we, m_i)
        gi = sload(wg, m_i)
        hb = sload(wh, m_i)
        ly_ = sload(wly, m_i)
        lx_ = sload(wlx, m_i)
        cutoff = ti - DELTA_T

        def cond(Lc):
            return sload(wt, Lc) < cutoff

        def expire(Lc):
            plsc.addupdate_scatter(
                g_v, [jnp.full((16,), sload(wg, Lc), jnp.int32)],
                jnp.full((16,), -sload(we, Lc), jnp.float32), mask=lane0)
            return Lc + 1

        L = lax.while_loop(cond, expire, L)

        plsc.addupdate_scatter(
            g_v, [jnp.full((16,), gi, jnp.int32)],
            jnp.full((16,), ei, jnp.float32), mask=lane0)

        inv = ones16 / jnp.full((16,), ei, jnp.float32)
        for k in range(4):
            valid = ((dy_t[k] >= R - ly_) & (dy_t[k] <= K + R - 1 - ly_) &
                     (dx_t[k] >= R - lx_) & (dx_t[k] <= K + R - 1 - lx_))
            idx = jnp.where(valid, gi + off_t[k], gi)
            g = plsc.load_gather(g_v, [idx])
            vals = g * jnp.where(valid, inv, zeros16)
            if k == 3:
                vals = vals + cnt_t
            plsc.addupdate(h_v.at[pl.ds(hb + 16 * k, 16)], vals)
        return L

    lax.fori_loop(0, nw, proc, jnp.int32(0))

    def norm(c, carry):
        cnt = (h_v[pl.ds(c * (2 * BIN) + 48, 16)][CNT_LANE - 48]
               + h_v[pl.ds(c * (2 * BIN) + BIN + 48, 16)][CNT_LANE - 48])
        scale = ones16 / jnp.full((16,), cnt + 1e-6, jnp.float32)
        for k in range(2 * BIN // 16):
            sl = pl.ds(c * (2 * BIN) + k * 16, 16)
            h_v[sl] = h_v[sl] * scale
        return carry

    lax.fori_loop(0, CPG, norm, 0)

    base = (b * NC + lo) * (2 * BIN)
    pltpu.sync_copy(h_v, out_hbm.at[pl.ds(base, HWORDS)])


@jax.jit
def _hats_sc(comp, len16):
    mesh = plsc.VectorSubcoreMesh(core_axis_name="c", subcore_axis_name="s",
                                  num_cores=2, num_subcores=16)
    f = pl.kernel(
        _body,
        out_type=jax.ShapeDtypeStruct((B * NC * 2 * BIN,), jnp.float32),
        mesh=mesh,
        compiler_params=pltpu.CompilerParams(needs_layout_passes=False),
        scratch_types=[
            pltpu.VMEM((4, TPAD), jnp.float32),
            pltpu.VMEM((16,), jnp.int32),
            pltpu.VMEM((GWORDS,), jnp.float32),
            pltpu.VMEM((HWORDS,), jnp.float32),
            pltpu.VMEM((WCAP,), jnp.float32),
            pltpu.VMEM((WCAP,), jnp.float32),
            pltpu.VMEM((WCAP,), jnp.int32),
            pltpu.VMEM((WCAP,), jnp.int32),
            pltpu.VMEM((WCAP,), jnp.int32),
            pltpu.VMEM((WCAP,), jnp.int32),
        ],
    )
    return f(comp, len16)


def kernel(events, lengths):
    comp = jnp.transpose(events, (0, 2, 1))
    len16 = jnp.zeros((16,), jnp.int32).at[:B].set(lengths.astype(jnp.int32))
    flat = _hats_sc(comp, len16)
    out = flat.reshape(B, NC, 2, BIN)[..., :S * S]
    return out.reshape(B, NC, 2, S, S)
